# baseline (speedup 1.0000x reference)
import functools
import jax
import jax.numpy as jnp
from jax import lax
from jax.experimental import pallas as pl
from jax.experimental.pallas import tpu as pltpu
from jax.experimental.pallas import tpu_sc as plsc

_B, _N, _M, _D = 8, 2048, 2048, 3
_L = 16
_PTS = 1024

_GDN = lax.GatherDimensionNumbers(
    offset_dims=(), collapsed_slice_dims=(0,), start_index_map=(0,))


def _rot(v, idx):
    return lax.gather(v, idx[:, None], dimension_numbers=_GDN,
                      slice_sizes=(1,),
                      mode=lax.GatherScatterMode.PROMISE_IN_BOUNDS)


def _sc_chamfer():
    mesh = plsc.VectorSubcoreMesh(core_axis_name="c", subcore_axis_name="s")

    @functools.partial(
        pl.kernel, mesh=mesh,
        out_type=[
            jax.ShapeDtypeStruct((_B, _N), jnp.float32),
            jax.ShapeDtypeStruct((_B, _M), jnp.float32),
        ],
        scratch_types=[
            pltpu.VMEM((_D, _N), jnp.float32),
            pltpu.VMEM((_D, _M), jnp.float32),
            pltpu.VMEM((_PTS,), jnp.float32),
        ],
    )
    def k(x_hbm, y_hbm, d1_hbm, d2_hbm, qv, cv, outv):
        wid = lax.axis_index("s") * 2 + lax.axis_index("c")
        half = wid // _L
        w = wid % _L
        b = w // 2
        q0 = (w % 2) * _PTS

        inf16 = jnp.full((_L,), jnp.float32(jnp.inf))
        iota = lax.iota(jnp.int32, _L)
        idxs = [(iota + r) & (_L - 1) for r in range(1, _L)]

        def run(src_q, src_c, dst):
            pltpu.sync_copy(src_q.at[b], qv)
            pltpu.sync_copy(src_c.at[b], cv)

            def point_chunk(ic, _):
                base = q0 + ic * _L
                q = [qv[kd, pl.ds(base, _L)] for kd in range(_D)]

                def cand_chunk(jc, acc):
                    j0 = jc * _L
                    c = [cv[kd, pl.ds(j0, _L)] for kd in range(_D)]
                    for r in range(_L):
                        d = None
                        for kd in range(_D):
                            cr = c[kd] if r == 0 else _rot(c[kd], idxs[r - 1])
                            t = q[kd] - cr
                            d = t * t if d is None else d + t * t
                        acc = jnp.minimum(acc, d)
                    return acc

                acc = lax.fori_loop(0, _M // _L, cand_chunk, inf16)
                outv[pl.ds(ic * _L, _L)] = acc
                return 0

            lax.fori_loop(0, _PTS // _L, point_chunk, 0)
            pltpu.sync_copy(outv, dst.at[b, pl.ds(q0, _PTS)])

        @pl.when(half == 0)
        def _dist1():
            run(x_hbm, y_hbm, d1_hbm)

        @pl.when(half == 1)
        def _dist2():
            run(y_hbm, x_hbm, d2_hbm)

    return k


@jax.jit
def kernel(input1, input2):
    x = jnp.transpose(input1, (0, 2, 1))
    y = jnp.transpose(input2, (0, 2, 1))
    d1, d2 = _sc_chamfer()(x, y)
    return (d1, d2)


# speedup vs baseline: 8.5907x; 8.5907x over previous
import jax
import jax.numpy as jnp
from jax.experimental import pallas as pl

_B, _N, _M, _D = 8, 2048, 2048, 3
_BB = 2


def _bf(a):
    return a.astype(jnp.bfloat16).astype(jnp.float32)


def _chamfer_batch(x_ref, y_ref, xr_ref, d1_ref, d2_ref):
    for b in range(_BB):
        xb = x_ref[b]
        yb = y_ref[b]
        xr = xr_ref[b]
        ny = jnp.sum(yb * yb, axis=0, keepdims=True)
        y2 = -2.0 * yb

        hx = _bf(xb)
        lx = _bf(xb - hx)
        hy = _bf(y2)
        ly = _bf(y2 - hy)
        nyh = _bf(ny)
        nyl = _bf(ny - nyh)
        nyll = _bf(ny - nyh - nyl)
        ones_n = jnp.ones((3, _N), jnp.float32)

        lhs = jnp.concatenate(
            [hx, hx, lx, ones_n],
            axis=0).astype(jnp.bfloat16)
        rhs = jnp.concatenate(
            [hy, ly, hy, nyh, nyl, nyll],
            axis=0).astype(jnp.bfloat16)
        e = jax.lax.dot_general(
            lhs, rhs, dimension_numbers=(((0,), (0,)), ((), ())),
            preferred_element_type=jnp.float32)
        nxc = jnp.sum(xr * xr, axis=1, keepdims=True)
        d1_ref[b] = jnp.min(e, axis=1, keepdims=True) + nxc
        d2_ref[b, 0, :] = jnp.min(e + nxc, axis=0)


@jax.jit
def kernel(input1, input2):
    x = jnp.transpose(input1, (0, 2, 1))
    y = jnp.transpose(input2, (0, 2, 1))
    d1, d2 = pl.pallas_call(
        _chamfer_batch,
        grid=(_B // _BB,),
        in_specs=[
            pl.BlockSpec((_BB, _D, _N), lambda b: (b, 0, 0)),
            pl.BlockSpec((_BB, _D, _M), lambda b: (b, 0, 0)),
            pl.BlockSpec((_BB, _N, _D), lambda b: (b, 0, 0)),
        ],
        out_specs=[
            pl.BlockSpec((_BB, _N, 1), lambda b: (b, 0, 0)),
            pl.BlockSpec((_BB, 1, _M), lambda b: (b, 0, 0)),
        ],
        out_shape=[
            jax.ShapeDtypeStruct((_B, _N, 1), jnp.float32),
            jax.ShapeDtypeStruct((_B, 1, _M), jnp.float32),
        ],
    )(x, y, input1)
    return (d1[:, :, 0], d2[:, 0, :])


# speedup vs baseline: 10.5655x; 1.2299x over previous
import jax
import jax.numpy as jnp
from jax.experimental import pallas as pl

_B, _N, _M, _D = 8, 2048, 2048, 3
_BB = 2


def _bf(a):
    return a.astype(jnp.bfloat16).astype(jnp.float32)


def _chamfer_batch(x_ref, y_ref, d1_ref, d2_ref):
    for b in range(_BB):
        xb = x_ref[b]
        yb = y_ref[b]
        nx = jnp.sum(xb * xb, axis=0, keepdims=True)
        ny = jnp.sum(yb * yb, axis=0, keepdims=True)
        y2 = -2.0 * yb

        hx = _bf(xb)
        lx = _bf(xb - hx)
        hy = _bf(y2)
        ly = _bf(y2 - hy)
        nxh = _bf(nx)
        nxl = _bf(nx - nxh)
        nxll = _bf(nx - nxh - nxl)
        nyh = _bf(ny)
        nyl = _bf(ny - nyh)
        nyll = _bf(ny - nyh - nyl)
        ones_n = jnp.ones((3, _N), jnp.float32)
        ones_m = jnp.ones((3, _M), jnp.float32)
        zeros_n = jnp.zeros((1, _N), jnp.float32)
        zeros_m = jnp.zeros((1, _M), jnp.float32)

        lhs = jnp.concatenate(
            [hx, hx, lx, nxh, nxl, nxll, ones_n, zeros_n],
            axis=0).astype(jnp.bfloat16)
        rhs = jnp.concatenate(
            [hy, ly, hy, ones_m, nyh, nyl, nyll, zeros_m],
            axis=0).astype(jnp.bfloat16)
        d = jax.lax.dot_general(
            lhs, rhs, dimension_numbers=(((0,), (0,)), ((), ())),
            preferred_element_type=jnp.float32)
        d1_ref[b] = jnp.min(d, axis=1, keepdims=True)
        d2_ref[b, 0, :] = jnp.min(d, axis=0)


@jax.jit
def kernel(input1, input2):
    x = jnp.transpose(input1, (0, 2, 1))
    y = jnp.transpose(input2, (0, 2, 1))
    d1, d2 = pl.pallas_call(
        _chamfer_batch,
        grid=(_B // _BB,),
        in_specs=[
            pl.BlockSpec((_BB, _D, _N), lambda b: (b, 0, 0)),
            pl.BlockSpec((_BB, _D, _M), lambda b: (b, 0, 0)),
        ],
        out_specs=[
            pl.BlockSpec((_BB, _N, 1), lambda b: (b, 0, 0)),
            pl.BlockSpec((_BB, 1, _M), lambda b: (b, 0, 0)),
        ],
        out_shape=[
            jax.ShapeDtypeStruct((_B, _N, 1), jnp.float32),
            jax.ShapeDtypeStruct((_B, 1, _M), jnp.float32),
        ],
    )(x, y)
    return (d1[:, :, 0], d2[:, 0, :])


# speedup vs baseline: 10.5775x; 1.0011x over previous
import jax
import jax.numpy as jnp
from jax.experimental import pallas as pl

_B, _N, _M, _D = 8, 2048, 2048, 3
_BB = 2


def _bf(a):
    return a.astype(jnp.bfloat16).astype(jnp.float32)


def _chamfer_batch(x_ref, y_ref, d1_ref, d2_ref):
    for b in range(_BB):
        xb = x_ref[b]
        yb = y_ref[b]
        nx = jnp.sum(xb * xb, axis=0, keepdims=True)
        ny = jnp.sum(yb * yb, axis=0, keepdims=True)
        y2 = -2.0 * yb

        hx = _bf(xb)
        lx = _bf(xb - hx)
        hy = _bf(y2)
        ly = _bf(y2 - hy)
        nxh = _bf(nx)
        nxl = _bf(nx - nxh)
        nxll = _bf(nx - nxh - nxl)
        nyh = _bf(ny)
        nyl = _bf(ny - nyh)
        nyll = _bf(ny - nyh - nyl)
        ones_n = jnp.ones((3, _N), jnp.float32)
        ones_m = jnp.ones((3, _M), jnp.float32)
        zeros_n = jnp.zeros((1, _N), jnp.float32)
        zeros_m = jnp.zeros((1, _M), jnp.float32)

        lhs = jnp.concatenate(
            [hx, hx, lx, nxh, nxl, nxll, ones_n, zeros_n],
            axis=0).astype(jnp.bfloat16)
        rhs = jnp.concatenate(
            [hy, ly, hy, ones_m, nyh, nyl, nyll, zeros_m],
            axis=0).astype(jnp.bfloat16)
        h = _M // 2
        da = jax.lax.dot_general(
            lhs, rhs[:, :h], dimension_numbers=(((0,), (0,)), ((), ())),
            preferred_element_type=jnp.float32)
        db = jax.lax.dot_general(
            lhs, rhs[:, h:], dimension_numbers=(((0,), (0,)), ((), ())),
            preferred_element_type=jnp.float32)
        d1a = jnp.min(da, axis=1, keepdims=True)
        d1b = jnp.min(db, axis=1, keepdims=True)
        d1_ref[b] = jnp.minimum(d1a, d1b)
        d2_ref[b, 0, :h] = jnp.min(da, axis=0)
        d2_ref[b, 0, h:] = jnp.min(db, axis=0)


@jax.jit
def kernel(input1, input2):
    x = jnp.transpose(input1, (0, 2, 1))
    y = jnp.transpose(input2, (0, 2, 1))
    d1, d2 = pl.pallas_call(
        _chamfer_batch,
        grid=(_B // _BB,),
        in_specs=[
            pl.BlockSpec((_BB, _D, _N), lambda b: (b, 0, 0)),
            pl.BlockSpec((_BB, _D, _M), lambda b: (b, 0, 0)),
        ],
        out_specs=[
            pl.BlockSpec((_BB, _N, 1), lambda b: (b, 0, 0)),
            pl.BlockSpec((_BB, 1, _M), lambda b: (b, 0, 0)),
        ],
        out_shape=[
            jax.ShapeDtypeStruct((_B, _N, 1), jnp.float32),
            jax.ShapeDtypeStruct((_B, 1, _M), jnp.float32),
        ],
    )(x, y)
    return (d1[:, :, 0], d2[:, 0, :])
